# Initial kernel scaffold; baseline (speedup 1.0000x reference)
import jax
import jax.numpy as jnp
from jax.experimental import pallas as pl


def kernel(x, edge_index, batch, exp_embedding, exp_bias, W1, b1, W2, b2):
    raise NotImplementedError("write your pallas kernel here")


# speedup vs baseline: 30.6668x; 30.6668x over previous
import functools

import jax
import jax.numpy as jnp
from jax import lax
from jax.experimental import pallas as pl
from jax.experimental.pallas import tpu as pltpu
from jax.experimental.pallas import tpu_sc as plsc

N = 50000
C = 16
E = 1600000
K = 10
ALPHA = 0.1
NGEN = 10000
NBATCH = 5

NP = 50176
NPT = NP // 16
UC = NPT // 8
NUC = 8

ERP = 12544
EP = ERP * 128
RPT = ERP // 16

_f32 = jnp.float32
_i32 = jnp.int32


_RPW = ERP // 32


def _deg_body(dst_ref, out_ref, sp_deg, dstb, ones_buf, zbuf):
    c = lax.axis_index("c")
    s = lax.axis_index("s")
    wg = c * 16 + s
    nbase = s * NPT
    zeros16 = jnp.zeros((16,), _f32)
    ones16 = jnp.ones((16,), _f32)

    @pl.loop(0, UC)
    def _zz(i):
        zbuf[i, :] = zeros16

    @pl.loop(0, 128)
    def _zo(i):
        ones_buf[i, :] = ones16

    @pl.loop(0, NUC)
    def _zs(ci):
        pltpu.sync_copy(zbuf, sp_deg.at[pl.ds(nbase + ci * UC, UC)])

    plsc.subcore_barrier()

    def _edge_block(row0, nrows):
        pltpu.sync_copy(dst_ref.at[pl.ds(row0, nrows)], dstb.at[pl.ds(0, nrows)])

        @pl.loop(0, nrows)
        def _j(j):
            pltpu.sync_copy(ones_buf, sp_deg.at[dstb.at[j]], add=True)

    row_base = wg * _RPW

    @pl.loop(0, 24)
    def _b(bi):
        _edge_block(row_base + bi * 16, 16)

    _edge_block(row_base + 24 * 16, 8)
    plsc.subcore_barrier()
    pltpu.sync_copy(sp_deg.at[pl.ds(nbase, NPT)],
                    out_ref.at[c, pl.ds(nbase, NPT)])


_SC_PARAMS = pltpu.CompilerParams(use_tc_tiling_on_sc=False)

_deg_call = pl.kernel(
    _deg_body,
    out_type=jax.ShapeDtypeStruct((2, NP, C), _f32),
    mesh=plsc.VectorSubcoreMesh(core_axis_name="c", subcore_axis_name="s"),
    compiler_params=_SC_PARAMS,
    scratch_types=[
        pltpu.VMEM_SHARED((NP, C), _f32),
        pltpu.VMEM((16, 128), _i32),
        pltpu.VMEM((128, C), _f32),
        pltpu.VMEM((UC, C), _f32),
    ],
)


def _prep_body(x_ref, emb_ref, bias_ref, degp_ref, w1_ref, b1_ref, w2_ref,
               b2_ref, t0_ref, q_ref, g_ref, sc_ref):
    h = emb_ref[...] * x_ref[...] + bias_ref[...]
    h = lax.dot_general(h, w1_ref[...], (((1,), (1,)), ((), ())),
                        preferred_element_type=_f32) + b1_ref[...]
    h = jnp.maximum(h, 0.0)
    h = lax.dot_general(h, w2_ref[...], (((1,), (1,)), ((), ())),
                        preferred_element_type=_f32) + b2_ref[...]
    deg = degp_ref[:, 0:1] + degp_ref[:, 1:2] + 1.0
    dinv = lax.rsqrt(deg)
    sq = dinv * deg
    t0_ref[...] = dinv * h
    q_ref[...] = (ALPHA / (1.0 - ALPHA)) * h * sq
    g_ref[...] = jnp.broadcast_to((1.0 - ALPHA) * dinv * dinv, h.shape)
    sc_ref[...] = jnp.broadcast_to(sq * (1.0 / NGEN), h.shape)


_PREP_BLK = 2000

_prep_call = pl.pallas_call(
    _prep_body,
    grid=(N // _PREP_BLK,),
    in_specs=[
        pl.BlockSpec((_PREP_BLK, 1), lambda i: (i, 0)),
        pl.BlockSpec((_PREP_BLK, C), lambda i: (i % 5, 0)),
        pl.BlockSpec((_PREP_BLK, 1), lambda i: (i % 5, 0)),
        pl.BlockSpec((_PREP_BLK, 2), lambda i: (i, 0)),
        pl.BlockSpec((C, C), lambda i: (0, 0)),
        pl.BlockSpec((1, C), lambda i: (0, 0)),
        pl.BlockSpec((C, C), lambda i: (0, 0)),
        pl.BlockSpec((1, C), lambda i: (0, 0)),
    ],
    out_specs=[
        pl.BlockSpec((_PREP_BLK, C), lambda i: (i, 0)),
        pl.BlockSpec((_PREP_BLK, C), lambda i: (i, 0)),
        pl.BlockSpec((_PREP_BLK, C), lambda i: (i, 0)),
        pl.BlockSpec((_PREP_BLK, C), lambda i: (i, 0)),
    ],
    out_shape=[
        jax.ShapeDtypeStruct((N, C), _f32),
        jax.ShapeDtypeStruct((N, C), _f32),
        jax.ShapeDtypeStruct((N, C), _f32),
        jax.ShapeDtypeStruct((N, C), _f32),
    ],
)


def _main_body(src_ref, dst_ref, t0_ref, q_ref, g_ref, sc_ref, out_ref,
               pool_ref, sp_t, sp_s, srcb, dstb, rows_buf,
               s_chunk, t_chunk, g_chunk, acc, pool_tmp, gsem):
    w = lax.axis_index("s")
    nbase = w * NPT
    row_base = w * RPT
    zeros16 = jnp.zeros((16,), _f32)

    @pl.loop(0, 8)
    def _za(i):
        acc[i, :] = zeros16

    @pl.loop(0, NUC)
    def _stage(ci):
        r0 = nbase + ci * UC
        pltpu.sync_copy(t0_ref.at[pl.ds(r0, UC)], t_chunk)
        pltpu.sync_copy(t_chunk, sp_t.at[pl.ds(r0, UC)])
        pltpu.sync_copy(q_ref.at[pl.ds(r0, UC)], s_chunk)
        pltpu.sync_copy(s_chunk, sp_s.at[pl.ds(r0, UC)])

    plsc.subcore_barrier()

    def _edge_block(row0, nrows):
        pltpu.sync_copy(src_ref.at[pl.ds(row0, nrows)], srcb.at[pl.ds(0, nrows)])
        pltpu.sync_copy(dst_ref.at[pl.ds(row0, nrows)], dstb.at[pl.ds(0, nrows)])

        @pl.loop(0, nrows)
        def _j(j):
            pltpu.async_copy(sp_t.at[srcb.at[j]], rows_buf, gsem).wait()
            pltpu.sync_copy(rows_buf, sp_s.at[dstb.at[j]], add=True)

    @pl.loop(0, K)
    def _k(k):
        @pl.loop(0, 49)
        def _b(bi):
            _edge_block(row_base + bi * 16, 16)

        plsc.subcore_barrier()

        @pl.loop(0, NUC)
        def _u(ci):
            r0 = nbase + ci * UC
            pltpu.sync_copy(sp_s.at[pl.ds(r0, UC)], s_chunk)
            pltpu.sync_copy(sp_t.at[pl.ds(r0, UC)], t_chunk)
            pltpu.sync_copy(g_ref.at[pl.ds(r0, UC)], g_chunk)

            @pl.loop(0, UC)
            def _r(i):
                t_chunk[i, :] = g_chunk[i, :] * (s_chunk[i, :] + t_chunk[i, :])

            pltpu.sync_copy(t_chunk, sp_t.at[pl.ds(r0, UC)])
            pltpu.sync_copy(q_ref.at[pl.ds(r0, UC)], s_chunk)
            pltpu.sync_copy(s_chunk, sp_s.at[pl.ds(r0, UC)])

        plsc.subcore_barrier()

    @pl.loop(0, NUC)
    def _p(ci):
        r0 = nbase + ci * UC
        pltpu.sync_copy(sp_t.at[pl.ds(r0, UC)], t_chunk)
        pltpu.sync_copy(sc_ref.at[pl.ds(r0, UC)], g_chunk)

        @pl.loop(0, UC)
        def _pr(i):
            b = lax.div(r0 + i, NGEN)
            acc[b, :] = acc[b, :] + t_chunk[i, :] * g_chunk[i, :]

    pltpu.sync_copy(acc, pool_ref.at[w])
    plsc.subcore_barrier()

    @pl.when(w == 0)
    def _out():
        @pl.loop(1, 16)
        def _pj(j):
            pltpu.sync_copy(pool_ref.at[j], pool_tmp)

            @pl.loop(0, 8)
            def _pa(i):
                acc[i, :] = acc[i, :] + pool_tmp[i, :]

        pltpu.sync_copy(acc.at[pl.ds(0, NBATCH)], out_ref)


_main_call = pl.kernel(
    _main_body,
    out_type=(jax.ShapeDtypeStruct((NBATCH, C), _f32),
              jax.ShapeDtypeStruct((16, 8, C), _f32)),
    mesh=plsc.VectorSubcoreMesh(core_axis_name="c", subcore_axis_name="s",
                                num_cores=1),
    compiler_params=_SC_PARAMS,
    scratch_types=[
        pltpu.VMEM_SHARED((NP, C), _f32),
        pltpu.VMEM_SHARED((NP, C), _f32),
        pltpu.VMEM((16, 128), _i32),
        pltpu.VMEM((16, 128), _i32),
        pltpu.VMEM((128, C), _f32),
        pltpu.VMEM((UC, C), _f32),
        pltpu.VMEM((UC, C), _f32),
        pltpu.VMEM((UC, C), _f32),
        pltpu.VMEM((8, C), _f32),
        pltpu.VMEM((8, C), _f32),
        pltpu.SemaphoreType.DMA,
    ],
)


def kernel(x, edge_index, batch, exp_embedding, exp_bias, W1, b1, W2, b2):
    del batch
    pad_idx = jnp.full((EP - E,), NP - 1, _i32)
    src2d = jnp.concatenate([edge_index[0], pad_idx]).reshape(ERP, 128)
    dst2d = jnp.concatenate([edge_index[1], pad_idx]).reshape(ERP, 128)
    deg2 = _deg_call(dst2d)
    degp = jnp.stack([deg2[0, :N, 0], deg2[1, :N, 0]], axis=1)
    t0, q, g, sc = _prep_call(x, exp_embedding, exp_bias, degp, W1,
                              b1.reshape(1, C), W2, b2.reshape(1, C))
    pad2 = ((0, NP - N), (0, 0))
    t0p = jnp.pad(t0, pad2)
    qp = jnp.pad(q, pad2)
    gp = jnp.pad(g, pad2)
    scp = jnp.pad(sc, pad2)
    pooled, _ = _main_call(src2d, dst2d, t0p, qp, gp, scp)
    return pooled


# speedup vs baseline: 51.0735x; 1.6654x over previous
import functools

import jax
import jax.numpy as jnp
from jax import lax
from jax.experimental import pallas as pl
from jax.experimental.pallas import tpu as pltpu
from jax.experimental.pallas import tpu_sc as plsc

N = 50000
C = 16
E = 1600000
K = 10
ALPHA = 0.1
NGEN = 10000
NBATCH = 5

NP = 50176
NPT = NP // 16
UC = NPT // 14
NUC = 14

ERP = 12544
EP = ERP * 128
RPT = ERP // 16

_f32 = jnp.float32
_i32 = jnp.int32


_RPW = ERP // 32


def _deg_body(dst_ref, out_ref, sp_deg, dstb, ones_buf, zbuf):
    c = lax.axis_index("c")
    s = lax.axis_index("s")
    wg = c * 16 + s
    nbase = s * NPT
    zeros16 = jnp.zeros((16,), _f32)
    ones16 = jnp.ones((16,), _f32)

    @pl.loop(0, UC)
    def _zz(i):
        zbuf[i, :] = zeros16

    @pl.loop(0, 128)
    def _zo(i):
        ones_buf[i, :] = ones16

    @pl.loop(0, NUC)
    def _zs(ci):
        pltpu.sync_copy(zbuf, sp_deg.at[pl.ds(nbase + ci * UC, UC)])

    plsc.subcore_barrier()

    def _edge_block(row0, nrows):
        pltpu.sync_copy(dst_ref.at[pl.ds(row0, nrows)], dstb.at[pl.ds(0, nrows)])

        @pl.loop(0, nrows)
        def _j(j):
            pltpu.sync_copy(ones_buf, sp_deg.at[dstb.at[j]], add=True)

    row_base = wg * _RPW

    @pl.loop(0, 24)
    def _b(bi):
        _edge_block(row_base + bi * 16, 16)

    _edge_block(row_base + 24 * 16, 8)
    plsc.subcore_barrier()
    pltpu.sync_copy(sp_deg.at[pl.ds(nbase, NPT)],
                    out_ref.at[c, pl.ds(nbase, NPT)])


_SC_PARAMS = pltpu.CompilerParams(use_tc_tiling_on_sc=False)

_deg_call = pl.kernel(
    _deg_body,
    out_type=jax.ShapeDtypeStruct((2, NP, C), _f32),
    mesh=plsc.VectorSubcoreMesh(core_axis_name="c", subcore_axis_name="s"),
    compiler_params=_SC_PARAMS,
    scratch_types=[
        pltpu.VMEM_SHARED((NP, C), _f32),
        pltpu.VMEM((16, 128), _i32),
        pltpu.VMEM((128, C), _f32),
        pltpu.VMEM((UC, C), _f32),
    ],
)


def _prep_body(x_ref, emb_ref, bias_ref, degp_ref, w1_ref, b1_ref, w2_ref,
               b2_ref, t0_ref, q_ref, g_ref, sc_ref):
    h = emb_ref[...] * x_ref[...] + bias_ref[...]
    h = lax.dot_general(h, w1_ref[...], (((1,), (1,)), ((), ())),
                        preferred_element_type=_f32) + b1_ref[...]
    h = jnp.maximum(h, 0.0)
    h = lax.dot_general(h, w2_ref[...], (((1,), (1,)), ((), ())),
                        preferred_element_type=_f32) + b2_ref[...]
    deg = degp_ref[:, 0:1] + degp_ref[:, 1:2] + 1.0
    dinv = lax.rsqrt(deg)
    sq = dinv * deg
    t0_ref[...] = dinv * h
    q_ref[...] = (ALPHA / (1.0 - ALPHA)) * h * sq
    g_ref[...] = jnp.broadcast_to((1.0 - ALPHA) * dinv * dinv, h.shape)
    sc_ref[...] = jnp.broadcast_to(sq * (1.0 / NGEN), h.shape)


_PREP_BLK = 2000

_prep_call = pl.pallas_call(
    _prep_body,
    grid=(N // _PREP_BLK,),
    in_specs=[
        pl.BlockSpec((_PREP_BLK, 1), lambda i: (i, 0)),
        pl.BlockSpec((_PREP_BLK, C), lambda i: (i % 5, 0)),
        pl.BlockSpec((_PREP_BLK, 1), lambda i: (i % 5, 0)),
        pl.BlockSpec((_PREP_BLK, 2), lambda i: (i, 0)),
        pl.BlockSpec((C, C), lambda i: (0, 0)),
        pl.BlockSpec((1, C), lambda i: (0, 0)),
        pl.BlockSpec((C, C), lambda i: (0, 0)),
        pl.BlockSpec((1, C), lambda i: (0, 0)),
    ],
    out_specs=[
        pl.BlockSpec((_PREP_BLK, C), lambda i: (i, 0)),
        pl.BlockSpec((_PREP_BLK, C), lambda i: (i, 0)),
        pl.BlockSpec((_PREP_BLK, C), lambda i: (i, 0)),
        pl.BlockSpec((_PREP_BLK, C), lambda i: (i, 0)),
    ],
    out_shape=[
        jax.ShapeDtypeStruct((N, C), _f32),
        jax.ShapeDtypeStruct((N, C), _f32),
        jax.ShapeDtypeStruct((N, C), _f32),
        jax.ShapeDtypeStruct((N, C), _f32),
    ],
)


def _main_body(src_ref, dst_ref, t0_ref, q_ref, g_ref, sc_ref, out_ref,
               pool_ref, sp_t, sp_s, srcb, dstb, rows_buf,
               s_chunk, t_chunk, g_chunk, acc, pool_tmp, gsem, ssem, isem):
    w = lax.axis_index("s")
    nbase = w * NPT
    row_base = w * RPT
    zeros16 = jnp.zeros((16,), _f32)

    @pl.loop(0, 8)
    def _za(i):
        acc[i, :] = zeros16

    @pl.loop(0, NUC)
    def _stage(ci):
        r0 = nbase + ci * UC
        pltpu.sync_copy(t0_ref.at[pl.ds(r0, UC)], t_chunk)
        pltpu.sync_copy(t_chunk, sp_t.at[pl.ds(r0, UC)])
        pltpu.sync_copy(q_ref.at[pl.ds(r0, UC)], s_chunk)
        pltpu.sync_copy(s_chunk, sp_s.at[pl.ds(r0, UC)])

    plsc.subcore_barrier()

    def _wait_gather(b):
        pltpu.make_async_copy(sp_t.at[srcb.at[0, 0]], rows_buf.at[b],
                              gsem.at[b]).wait()

    def _wait_scatter(b):
        pltpu.make_async_copy(rows_buf.at[b], sp_s.at[dstb.at[0, 0]],
                              ssem.at[b]).wait()

    def _wait_idx():
        pltpu.make_async_copy(src_ref.at[pl.ds(0, 16)], srcb.at[0], isem).wait()
        pltpu.make_async_copy(dst_ref.at[pl.ds(0, 16)], dstb.at[0], isem).wait()

    @pl.loop(0, K)
    def _k(k):
        pltpu.sync_copy(src_ref.at[pl.ds(row_base, 16)], srcb.at[0])
        pltpu.sync_copy(dst_ref.at[pl.ds(row_base, 16)], dstb.at[0])
        pltpu.async_copy(src_ref.at[pl.ds(row_base + 16, 16)], srcb.at[1], isem)
        pltpu.async_copy(dst_ref.at[pl.ds(row_base + 16, 16)], dstb.at[1], isem)
        pltpu.async_copy(sp_t.at[srcb.at[0, 0]], rows_buf.at[0], gsem.at[0])
        pltpu.async_copy(sp_t.at[srcb.at[0, 1]], rows_buf.at[1], gsem.at[1])

        @pl.loop(0, RPT)
        def _j(j):
            b = lax.rem(j, 4)
            blk = lax.div(j, 16)
            jj = lax.rem(j, 16)
            pp = lax.rem(blk, 2)
            _wait_gather(b)
            pltpu.async_copy(rows_buf.at[b], sp_s.at[dstb.at[pp, jj]],
                             ssem.at[b], add=True)

            @pl.when(jnp.logical_and(jj == 2, j < 771))
            def _issue_idx():
                pn = lax.rem(blk + 1, 2)
                r0n = row_base + (blk + 1) * 16
                pltpu.async_copy(src_ref.at[pl.ds(r0n, 16)], srcb.at[pn], isem)
                pltpu.async_copy(dst_ref.at[pl.ds(r0n, 16)], dstb.at[pn], isem)

            @pl.when(jnp.logical_and(jj == 14, j < 768))
            def _drain_idx():
                _wait_idx()

            @pl.when(j < RPT - 2)
            def _issue_gather():
                j2 = j + 2
                b2 = lax.rem(j2, 4)

                @pl.when(j >= 2)
                def _free_buf():
                    _wait_scatter(b2)

                pltpu.async_copy(
                    sp_t.at[srcb.at[lax.rem(lax.div(j2, 16), 2),
                                    lax.rem(j2, 16)]],
                    rows_buf.at[b2], gsem.at[b2])

        _wait_scatter(0)
        _wait_scatter(1)
        _wait_scatter(2)
        _wait_scatter(3)
        plsc.subcore_barrier()

        @pl.loop(0, NUC)
        def _u(ci):
            r0 = nbase + ci * UC
            pltpu.sync_copy(sp_s.at[pl.ds(r0, UC)], s_chunk)
            pltpu.sync_copy(sp_t.at[pl.ds(r0, UC)], t_chunk)
            pltpu.sync_copy(g_ref.at[pl.ds(r0, UC)], g_chunk)

            @pl.loop(0, UC)
            def _r(i):
                t_chunk[i, :] = g_chunk[i, :] * (s_chunk[i, :] + t_chunk[i, :])

            pltpu.sync_copy(t_chunk, sp_t.at[pl.ds(r0, UC)])
            pltpu.sync_copy(q_ref.at[pl.ds(r0, UC)], s_chunk)
            pltpu.sync_copy(s_chunk, sp_s.at[pl.ds(r0, UC)])

        plsc.subcore_barrier()

    @pl.loop(0, NUC)
    def _p(ci):
        r0 = nbase + ci * UC
        pltpu.sync_copy(sp_t.at[pl.ds(r0, UC)], t_chunk)
        pltpu.sync_copy(sc_ref.at[pl.ds(r0, UC)], g_chunk)

        @pl.loop(0, UC)
        def _pr(i):
            b = lax.div(r0 + i, NGEN)
            acc[b, :] = acc[b, :] + t_chunk[i, :] * g_chunk[i, :]

    pltpu.sync_copy(acc, pool_ref.at[w])
    plsc.subcore_barrier()

    @pl.when(w == 0)
    def _out():
        @pl.loop(1, 16)
        def _pj(j):
            pltpu.sync_copy(pool_ref.at[j], pool_tmp)

            @pl.loop(0, 8)
            def _pa(i):
                acc[i, :] = acc[i, :] + pool_tmp[i, :]

        pltpu.sync_copy(acc.at[pl.ds(0, NBATCH)], out_ref)


_main_call = pl.kernel(
    _main_body,
    out_type=(jax.ShapeDtypeStruct((NBATCH, C), _f32),
              jax.ShapeDtypeStruct((16, 8, C), _f32)),
    mesh=plsc.VectorSubcoreMesh(core_axis_name="c", subcore_axis_name="s",
                                num_cores=1),
    compiler_params=_SC_PARAMS,
    scratch_types=[
        pltpu.VMEM_SHARED((NP, C), _f32),
        pltpu.VMEM_SHARED((NP, C), _f32),
        pltpu.VMEM((2, 16, 128), _i32),
        pltpu.VMEM((2, 16, 128), _i32),
        pltpu.VMEM((4, 128, C), _f32),
        pltpu.VMEM((UC, C), _f32),
        pltpu.VMEM((UC, C), _f32),
        pltpu.VMEM((UC, C), _f32),
        pltpu.VMEM((8, C), _f32),
        pltpu.VMEM((8, C), _f32),
        pltpu.SemaphoreType.DMA((4,)),
        pltpu.SemaphoreType.DMA((4,)),
        pltpu.SemaphoreType.DMA,
    ],
)


def kernel(x, edge_index, batch, exp_embedding, exp_bias, W1, b1, W2, b2):
    del batch
    pad_idx = jnp.full((EP - E,), NP - 1, _i32)
    src2d = jnp.concatenate([edge_index[0], pad_idx]).reshape(ERP, 128)
    dst2d = jnp.concatenate([edge_index[1], pad_idx]).reshape(ERP, 128)
    deg2 = _deg_call(dst2d)
    degp = jnp.stack([deg2[0, :N, 0], deg2[1, :N, 0]], axis=1)
    t0, q, g, sc = _prep_call(x, exp_embedding, exp_bias, degp, W1,
                              b1.reshape(1, C), W2, b2.reshape(1, C))
    pad2 = ((0, NP - N), (0, 0))
    t0p = jnp.pad(t0, pad2)
    qp = jnp.pad(q, pad2)
    gp = jnp.pad(g, pad2)
    scp = jnp.pad(sc, pad2)
    pooled, _ = _main_call(src2d, dst2d, t0p, qp, gp, scp)
    return pooled
